# baseline (speedup 1.0000x reference)
import functools

import jax
import jax.numpy as jnp
from jax import lax
from jax.experimental import pallas as pl
from jax.experimental.pallas import tpu as pltpu
from jax.experimental.pallas import tpu_sc as plsc

B = 4096
L = 200
E = 64
HID = 256
NOUT = 5

NC = 2
NS = 16
NW = NC * NS
BPW = B // NW

CH = 40
NCH = L // CH
NBUF = 4
UNROLL = 8
LANES = 16
EV = E // LANES


def _sc_pool_body(idx_hbm, emb_hbm, out_hbm, idx_v, rows_v, pooled_v,
                  s0, s1, s2, s3):
    sems = (s0, s1, s2, s3)
    wid = lax.axis_index("s") * NC + lax.axis_index("c")

    pltpu.sync_copy(idx_hbm.at[pl.ds(wid * BPW, BPW), :], idx_v)

    def issue(b, e_local):
        for k in range(NCH):
            pltpu.async_copy(
                emb_hbm.at[idx_v.at[e_local, pl.ds(k * CH, CH)]],
                rows_v.at[b, pl.ds(k * CH, CH), :],
                sems[b],
            )

    def drain(b):
        pltpu.make_async_copy(
            emb_hbm.at[pl.ds(0, L), :], rows_v.at[b], sems[b]
        ).wait()

    for b in range(NBUF):
        issue(b, b)

    zero = jnp.zeros((LANES,), jnp.float32)
    inv_l = jnp.float32(1.0 / L)

    @pl.loop(0, BPW // NBUF)
    def _group(g):
        for b in range(NBUF):
            e = g * NBUF + b
            drain(b)

            def acc_body(jv, accs):
                accs = list(accs)
                for u in range(UNROLL):
                    j = jv * UNROLL + u
                    for c in range(EV):
                        accs[c] = accs[c] + rows_v[b, j, pl.ds(c * LANES, LANES)]
                return tuple(accs)

            accs = lax.fori_loop(0, L // UNROLL, acc_body, (zero,) * EV)
            for c in range(EV):
                pooled_v[e, pl.ds(c * LANES, LANES)] = accs[c] * inv_l

            nxt = e + NBUF

            @pl.when(nxt < BPW)
            def _():
                issue(b, nxt)

    pltpu.sync_copy(pooled_v, out_hbm.at[pl.ds(wid * BPW, BPW), :])


@jax.jit
def _sc_pool(idx, emb):
    mesh = plsc.VectorSubcoreMesh(core_axis_name="c", subcore_axis_name="s")
    f = pl.kernel(
        _sc_pool_body,
        out_type=jax.ShapeDtypeStruct((B, E), jnp.float32),
        mesh=mesh,
        scratch_types=[
            pltpu.VMEM((BPW, L), jnp.int32),
            pltpu.VMEM((NBUF, L, E), jnp.float32),
            pltpu.VMEM((BPW, E), jnp.float32),
            pltpu.SemaphoreType.DMA,
            pltpu.SemaphoreType.DMA,
            pltpu.SemaphoreType.DMA,
            pltpu.SemaphoreType.DMA,
        ],
        compiler_params=pltpu.CompilerParams(use_tc_tiling_on_sc=False),
    )
    return f(idx, emb)


def _mlp_body(x_ref, w1_ref, b1_ref, w2_ref, b2_ref, o_ref):
    x = x_ref[...]
    h = jnp.dot(x, w1_ref[...], preferred_element_type=jnp.float32)
    h = jnp.maximum(h + b1_ref[...], 0.0)
    logits = jnp.dot(h, w2_ref[...], preferred_element_type=jnp.float32)
    logits = logits + b2_ref[...]
    shifted = logits - jnp.max(logits, axis=-1, keepdims=True)
    lse = jnp.log(jnp.sum(jnp.exp(shifted), axis=-1, keepdims=True))
    o_ref[...] = shifted - lse


@jax.jit
def _tc_mlp(pooled, W1, b1, W2, b2):
    return pl.pallas_call(
        _mlp_body,
        out_shape=jax.ShapeDtypeStruct((B, NOUT), jnp.float32),
    )(pooled, W1, b1.reshape(1, HID), W2, b2.reshape(1, NOUT))


def kernel(input, emb, W1, b1, W2, b2):
    pooled = _sc_pool(input, emb)
    return _tc_mlp(pooled, W1, b1, W2, b2)


# speedup vs baseline: 1.0006x; 1.0006x over previous
import functools

import jax
import jax.numpy as jnp
from jax import lax
from jax.experimental import pallas as pl
from jax.experimental.pallas import tpu as pltpu
from jax.experimental.pallas import tpu_sc as plsc

B = 4096
L = 200
E = 64
HID = 256
NOUT = 5

NC = 2
NS = 16
NW = NC * NS
BPW = B // NW

CH = 40
NCH = L // CH
NBUF = 4
UNROLL = 8
LANES = 16
EV = E // LANES


def _sc_pool_body(idx_hbm, emb_hbm, out_hbm, idx_v, rows_v, pooled_v,
                  s0, s1, s2, s3):
    sems = (s0, s1, s2, s3)
    wid = lax.axis_index("s") * NC + lax.axis_index("c")

    pltpu.sync_copy(idx_hbm.at[pl.ds(wid * BPW * L, BPW * L)], idx_v)

    def issue(b, e_local):
        for k in range(NCH):
            pltpu.async_copy(
                emb_hbm.at[idx_v.at[pl.ds(e_local * L + k * CH, CH)]],
                rows_v.at[b, pl.ds(k * CH, CH), :],
                sems[b],
            )

    def drain(b):
        pltpu.make_async_copy(
            emb_hbm.at[pl.ds(0, L), :], rows_v.at[b], sems[b]
        ).wait()

    for b in range(NBUF):
        issue(b, b)

    zero = jnp.zeros((LANES,), jnp.float32)
    inv_l = jnp.float32(1.0 / L)

    @pl.loop(0, BPW // NBUF)
    def _group(g):
        for b in range(NBUF):
            e = g * NBUF + b
            drain(b)

            def acc_body(jv, accs):
                accs = list(accs)
                for u in range(UNROLL):
                    j = jv * UNROLL + u
                    for c in range(EV):
                        accs[c] = accs[c] + rows_v[b, j, pl.ds(c * LANES, LANES)]
                return tuple(accs)

            accs = lax.fori_loop(0, L // UNROLL, acc_body, (zero,) * EV)
            for c in range(EV):
                pooled_v[e, pl.ds(c * LANES, LANES)] = accs[c] * inv_l

            nxt = e + NBUF

            @pl.when(nxt < BPW)
            def _():
                issue(b, nxt)

    pltpu.sync_copy(pooled_v, out_hbm.at[pl.ds(wid * BPW, BPW), :])


@jax.jit
def _sc_pool(idx, emb):
    mesh = plsc.VectorSubcoreMesh(core_axis_name="c", subcore_axis_name="s")
    f = pl.kernel(
        _sc_pool_body,
        out_type=jax.ShapeDtypeStruct((B, E), jnp.float32),
        mesh=mesh,
        scratch_types=[
            pltpu.VMEM((BPW * L,), jnp.int32),
            pltpu.VMEM((NBUF, L, E), jnp.float32),
            pltpu.VMEM((BPW, E), jnp.float32),
            pltpu.SemaphoreType.DMA,
            pltpu.SemaphoreType.DMA,
            pltpu.SemaphoreType.DMA,
            pltpu.SemaphoreType.DMA,
        ],
        compiler_params=pltpu.CompilerParams(use_tc_tiling_on_sc=False),
    )
    return f(idx, emb)


def _mlp_body(x_ref, w1_ref, b1_ref, w2_ref, b2_ref, o_ref):
    x = x_ref[...]
    h = jnp.dot(x, w1_ref[...], preferred_element_type=jnp.float32)
    h = jnp.maximum(h + b1_ref[...], 0.0)
    logits = jnp.dot(h, w2_ref[...], preferred_element_type=jnp.float32)
    logits = logits + b2_ref[...]
    shifted = logits - jnp.max(logits, axis=-1, keepdims=True)
    lse = jnp.log(jnp.sum(jnp.exp(shifted), axis=-1, keepdims=True))
    o_ref[...] = shifted - lse


@jax.jit
def _tc_mlp(pooled, W1, b1, W2, b2):
    return pl.pallas_call(
        _mlp_body,
        out_shape=jax.ShapeDtypeStruct((B, NOUT), jnp.float32),
    )(pooled, W1, b1.reshape(1, HID), W2, b2.reshape(1, NOUT))


def kernel(input, emb, W1, b1, W2, b2):
    pooled = _sc_pool(input.reshape(-1), emb)
    return _tc_mlp(pooled, W1, b1, W2, b2)


# speedup vs baseline: 1.3034x; 1.3026x over previous
import functools

import jax
import jax.numpy as jnp
from jax import lax
from jax.experimental import pallas as pl
from jax.experimental.pallas import tpu as pltpu
from jax.experimental.pallas import tpu_sc as plsc

B = 4096
L = 200
E = 64
HID = 256
NOUT = 5

NC = 2
NS = 16
NW = NC * NS
BPW = B // NW

NBUF = 2
UNROLL = 8
IUNROLL = 16
LANES = 16
EV = E // LANES


def _sc_pool_body(idx_hbm, emb_hbm, out_hbm, idx_v, rows_v, pooled_v,
                  s0, s1):
    sems = (s0, s1)
    wid = lax.axis_index("s") * NC + lax.axis_index("c")

    pltpu.sync_copy(idx_hbm.at[pl.ds(wid * BPW * L, BPW * L)], idx_v)

    def issue(b, e_local):
        def enq(vec, u, j):
            pltpu.async_copy(
                emb_hbm.at[vec[u], :],
                rows_v.at[b, j, :],
                sems[b],
            )

        @pl.loop(0, L // IUNROLL)
        def _rows(jv):
            vec = idx_v[pl.ds(e_local * L + jv * IUNROLL, IUNROLL)]
            for u in range(IUNROLL):
                enq(vec, u, jv * IUNROLL + u)

        tail = L % IUNROLL
        if tail:
            vec = idx_v[pl.ds(e_local * L + L - IUNROLL, IUNROLL)]
            for u in range(IUNROLL - tail, IUNROLL):
                enq(vec, u, L - IUNROLL + u)

    def drain(b):
        pltpu.make_async_copy(
            emb_hbm.at[pl.ds(0, L), :], rows_v.at[b], sems[b]
        ).wait()

    for b in range(NBUF):
        issue(b, b)

    zero = jnp.zeros((LANES,), jnp.float32)
    inv_l = jnp.float32(1.0 / L)

    @pl.loop(0, BPW // NBUF)
    def _group(g):
        for b in range(NBUF):
            e = g * NBUF + b
            drain(b)

            def acc_body(jv, accs):
                accs = list(accs)
                for u in range(UNROLL):
                    j = jv * UNROLL + u
                    for c in range(EV):
                        accs[c] = accs[c] + rows_v[b, j, pl.ds(c * LANES, LANES)]
                return tuple(accs)

            accs = lax.fori_loop(0, L // UNROLL, acc_body, (zero,) * EV)
            for c in range(EV):
                pooled_v[pl.ds(e * E + c * LANES, LANES)] = accs[c] * inv_l

            nxt = e + NBUF

            @pl.when(nxt < BPW)
            def _():
                issue(b, nxt)

    pltpu.sync_copy(pooled_v, out_hbm.at[pl.ds(wid * BPW * E, BPW * E)])


@jax.jit
def _sc_pool(idx, emb):
    mesh = plsc.VectorSubcoreMesh(core_axis_name="c", subcore_axis_name="s")
    f = pl.kernel(
        _sc_pool_body,
        out_type=jax.ShapeDtypeStruct((B * E,), jnp.float32),
        mesh=mesh,
        scratch_types=[
            pltpu.VMEM((BPW * L,), jnp.int32),
            pltpu.VMEM((NBUF, L, E), jnp.float32),
            pltpu.VMEM((BPW * E,), jnp.float32),
            pltpu.SemaphoreType.DMA,
            pltpu.SemaphoreType.DMA,
        ],
        compiler_params=pltpu.CompilerParams(use_tc_tiling_on_sc=True),
    )
    return f(idx, emb)


def _mlp_body(x_ref, w1_ref, b1_ref, w2_ref, b2_ref, o_ref):
    x = x_ref[...]
    h = jnp.dot(x, w1_ref[...], preferred_element_type=jnp.float32)
    h = jnp.maximum(h + b1_ref[...], 0.0)
    logits = jnp.dot(h, w2_ref[...], preferred_element_type=jnp.float32)
    logits = logits + b2_ref[...]
    shifted = logits - jnp.max(logits, axis=-1, keepdims=True)
    lse = jnp.log(jnp.sum(jnp.exp(shifted), axis=-1, keepdims=True))
    o_ref[...] = shifted - lse


@jax.jit
def _tc_mlp(pooled, W1, b1, W2, b2):
    return pl.pallas_call(
        _mlp_body,
        out_shape=jax.ShapeDtypeStruct((B, NOUT), jnp.float32),
    )(pooled, W1, b1.reshape(1, HID), W2, b2.reshape(1, NOUT))


def kernel(input, emb, W1, b1, W2, b2):
    pooled = _sc_pool(input.reshape(-1), emb).reshape(B, E)
    return _tc_mlp(pooled, W1, b1, W2, b2)
